# Initial kernel scaffold; baseline (speedup 1.0000x reference)
import jax
import jax.numpy as jnp
from jax.experimental import pallas as pl


def kernel(embeds, edge_weight, edge_index, u_idx, v_idx):
    raise NotImplementedError("write your pallas kernel here")


# speedup vs baseline: 8.8011x; 8.8011x over previous
import functools

import jax
import jax.numpy as jnp
from jax import lax
from jax.experimental import pallas as pl
from jax.experimental.pallas import tpu as pltpu
from jax.experimental.pallas import tpu_sc as plsc

N_NODES = 10000
D = 128
N_EDGES = 320000
N_BATCH = 16384
NC = 2
NS = 16
NW = NC * NS
LANES = 16

CHUNK = 64
CHUNKS_PER_W = 160
NRB = 4
NEB = 10
N_EDGES_PAD = NW * CHUNKS_PER_W * CHUNK
ROWS_PER_TILE = 624
ROWS_TAIL = N_NODES - NS * ROWS_PER_TILE

PAIRS_PER_W = N_BATCH // NW
SCHUNK = 128
N_SCHUNKS = PAIRS_PER_W // SCHUNK

_mesh = plsc.VectorSubcoreMesh(core_axis_name="c", subcore_axis_name="s")

_DNUMS = lax.GatherDimensionNumbers(
    offset_dims=(), collapsed_slice_dims=(0,), start_index_map=(0,))


def _dg(vec, idx):
    return lax.gather(vec, idx[:, None], _DNUMS, (1,),
                      mode=lax.GatherScatterMode.PROMISE_IN_BOUNDS)


def _lane_sum(v, iota):
    for k in (8, 4, 2, 1):
        v = v + _dg(v, iota ^ k)
    return v


@functools.partial(
    pl.kernel,
    out_type=jax.ShapeDtypeStruct((NC, N_NODES, D), jnp.float32),
    mesh=_mesh,
    scratch_types=[
        pltpu.VMEM_SHARED((N_NODES, D), jnp.float32),
        [pltpu.VMEM((2, CHUNK), jnp.int32) for _ in range(NEB)],
        [pltpu.VMEM((CHUNK,), jnp.float32) for _ in range(NEB)],
        [pltpu.VMEM((CHUNK, D), jnp.float32) for _ in range(NRB)],
        [pltpu.SemaphoreType.DMA for _ in range(NEB)],
        [pltpu.SemaphoreType.DMA for _ in range(NRB)],
        [pltpu.SemaphoreType.DMA for _ in range(NRB)],
    ],
)
def _propagate(h_hbm, epk_hbm, wpk_hbm, zero_hbm, out_hbm,
               acc, ebufs, wbufs, rbufs, esems, gsems, ssems):
    c = lax.axis_index("c")
    s = lax.axis_index("s")
    wid = s * NC + c
    cb = wid * CHUNKS_PER_W

    def start_edge(eb, j):
        pltpu.async_copy(epk_hbm.at[j], ebufs[eb], esems[eb])
        pltpu.async_copy(wpk_hbm.at[j], wbufs[eb], esems[eb])

    def wait_edge(eb):
        pltpu.make_async_copy(epk_hbm.at[0], ebufs[eb], esems[eb]).wait()
        pltpu.make_async_copy(wpk_hbm.at[0], wbufs[eb], esems[eb]).wait()

    def start_gather(rb, eb):
        return pltpu.async_copy(h_hbm.at[ebufs[eb].at[1]], rbufs[rb],
                                gsems[rb])

    def start_scatter(rb, eb):
        return pltpu.async_copy(rbufs[rb], acc.at[ebufs[eb].at[0]],
                                ssems[rb], add=True)

    def multiply(rb, eb):
        r_ref = rbufs[rb]
        w_ref = wbufs[eb]

        def mbody(fb, carry):
            w16 = w_ref[pl.ds(fb * LANES, LANES)]
            for e in range(LANES):
                we = _dg(w16, jnp.full((LANES,), e, jnp.int32))
                row = fb * LANES + e
                for f in range(D // LANES):
                    sl = pl.ds(f * LANES, LANES)
                    r_ref[row, sl] = r_ref[row, sl] * we
            return carry

        lax.fori_loop(0, CHUNK // LANES, mbody, 0)

    for q in range(NEB):
        start_edge(q, cb + q)
    row_lo = s * ROWS_PER_TILE
    pltpu.sync_copy(zero_hbm.at[pl.ds(row_lo, ROWS_PER_TILE)],
                    acc.at[pl.ds(row_lo, ROWS_PER_TILE)])

    @pl.when(s == 0)
    def _zero_tail():
        pltpu.sync_copy(zero_hbm.at[pl.ds(NS * ROWS_PER_TILE, ROWS_TAIL)],
                        acc.at[pl.ds(NS * ROWS_PER_TILE, ROWS_TAIL)])

    plsc.subcore_barrier()

    n_body = CHUNKS_PER_W // NEB

    def oct_body(t, carry):
        base = cb + NEB * t
        not_last = t < n_body - 1
        gd = {}
        sd = {}
        for q in (0, 1):
            wait_edge(q)
            gd[q] = start_gather(q % NRB, q)
        for q in range(NEB):
            if q >= 2:
                sd[q - 2].wait()

                @pl.when(not_last)
                def _prefetch(q=q):
                    start_edge(q - 2, base + NEB + q - 2)

            if q + 2 < NEB:
                wait_edge(q + 2)
                gd[q + 2] = start_gather((q + 2) % NRB, q + 2)
            gd[q].wait()
            multiply(q % NRB, q)
            sd[q] = start_scatter(q % NRB, q)
        sd[NEB - 2].wait()
        sd[NEB - 1].wait()

        @pl.when(not_last)
        def _prefetch_tail():
            start_edge(NEB - 2, base + 2 * NEB - 2)
            start_edge(NEB - 1, base + 2 * NEB - 1)

        return carry

    lax.fori_loop(0, n_body, oct_body, 0)
    plsc.subcore_barrier()
    pltpu.sync_copy(acc.at[pl.ds(row_lo, ROWS_PER_TILE)],
                    out_hbm.at[c, pl.ds(row_lo, ROWS_PER_TILE)])

    @pl.when(s == 0)
    def _write_tail():
        pltpu.sync_copy(acc.at[pl.ds(NS * ROWS_PER_TILE, ROWS_TAIL)],
                        out_hbm.at[c, pl.ds(NS * ROWS_PER_TILE, ROWS_TAIL)])


def _combine_body(p0_ref, p1_ref, s_ref, h_ref, snew_ref):
    h = p0_ref[...] + p1_ref[...]
    h_ref[...] = h
    snew_ref[...] = s_ref[...] + h


_combine = pl.pallas_call(
    _combine_body,
    grid=(10,),
    in_specs=[pl.BlockSpec((1000, D), lambda i: (i, 0))] * 3,
    out_specs=[pl.BlockSpec((1000, D), lambda i: (i, 0))] * 2,
    out_shape=[jax.ShapeDtypeStruct((N_NODES, D), jnp.float32)] * 2,
)


@functools.partial(
    pl.kernel,
    out_type=jax.ShapeDtypeStruct((N_BATCH,), jnp.float32),
    mesh=_mesh,
    scratch_types=[
        [pltpu.VMEM((SCHUNK,), jnp.int32) for _ in range(4)],
        [pltpu.VMEM((SCHUNK, D), jnp.float32) for _ in range(4)],
        [pltpu.VMEM((SCHUNK,), jnp.float32) for _ in range(2)],
        [pltpu.SemaphoreType.DMA for _ in range(2)],
        [pltpu.SemaphoreType.DMA for _ in range(4)],
        [pltpu.SemaphoreType.DMA for _ in range(2)],
    ],
)
def _score(sum_hbm, u_hbm, v_hbm, out_hbm,
           ibufs, rbufs, obufs, isems, gsems, osems):
    c = lax.axis_index("c")
    s = lax.axis_index("s")
    wid = s * NC + c
    tb = wid * PAIRS_PER_W

    def start_idx(b, j):
        base = tb + j * SCHUNK
        pltpu.async_copy(u_hbm.at[pl.ds(base, SCHUNK)], ibufs[2 * b], isems[b])
        pltpu.async_copy(v_hbm.at[pl.ds(base, SCHUNK)], ibufs[2 * b + 1],
                         isems[b])

    def wait_idx(b):
        pltpu.make_async_copy(u_hbm.at[pl.ds(0, SCHUNK)], ibufs[2 * b],
                              isems[b]).wait()
        pltpu.make_async_copy(v_hbm.at[pl.ds(0, SCHUNK)], ibufs[2 * b + 1],
                              isems[b]).wait()

    def compute(b, j):
        ur_v = rbufs[2 * b]
        vr_v = rbufs[2 * b + 1]
        o_v = obufs[b]
        iota = lax.broadcasted_iota(jnp.int32, (LANES,), 0)

        def pb_body(pb, carry):
            dots = jnp.zeros((LANES,), jnp.float32)
            for p in range(LANES):
                row = pb * LANES + p
                acc = ur_v[row, pl.ds(0, LANES)] * vr_v[row, pl.ds(0, LANES)]
                for f in range(1, D // LANES):
                    sl = pl.ds(f * LANES, LANES)
                    acc = acc + ur_v[row, sl] * vr_v[row, sl]
                tot = _lane_sum(acc, iota)
                dots = jnp.where(iota == p, tot, dots)
            x = dots * (1.0 / 16.0)
            o_v[pl.ds(pb * LANES, LANES)] = 1.0 + 4.0 / (1.0 + jnp.exp(-x))
            return carry

        lax.fori_loop(0, SCHUNK // LANES, pb_body, 0)
        return pltpu.async_copy(o_v, out_hbm.at[pl.ds(tb + j * SCHUNK, SCHUNK)],
                                osems[b])

    start_idx(0, 0)
    start_idx(1, 1)

    def body(t, carry):
        j0 = 2 * t
        wait_idx(0)
        g0a = pltpu.async_copy(sum_hbm.at[ibufs[0]], rbufs[0], gsems[0])
        g0b = pltpu.async_copy(sum_hbm.at[ibufs[1]], rbufs[1], gsems[1])
        wait_idx(1)
        g1a = pltpu.async_copy(sum_hbm.at[ibufs[2]], rbufs[2], gsems[2])
        g1b = pltpu.async_copy(sum_hbm.at[ibufs[3]], rbufs[3], gsems[3])
        g0a.wait()
        g0b.wait()
        o0 = compute(0, j0)

        @pl.when(t < N_SCHUNKS // 2 - 1)
        def _prefetch0():
            start_idx(0, j0 + 2)

        g1a.wait()
        g1b.wait()
        o1 = compute(1, j0 + 1)

        @pl.when(t < N_SCHUNKS // 2 - 1)
        def _prefetch1():
            start_idx(1, j0 + 3)

        o0.wait()
        o1.wait()
        return carry

    lax.fori_loop(0, N_SCHUNKS // 2, body, 0)


def kernel(embeds, edge_weight, edge_index, u_idx, v_idx):
    dst = edge_index[0]
    src = edge_index[1]
    pad = N_EDGES_PAD - N_EDGES
    padidx = jnp.arange(pad, dtype=jnp.int32) % N_NODES
    dstp = jnp.concatenate([dst, padidx])
    srcp = jnp.concatenate([src, padidx])
    wp = jnp.concatenate([edge_weight, jnp.zeros((pad,), jnp.float32)])
    wpk = wp.reshape(NW * CHUNKS_PER_W, CHUNK)
    epk = jnp.stack([dstp, srcp])
    epk = epk.reshape(2, NW * CHUNKS_PER_W, CHUNK).transpose(1, 0, 2)
    zeros = jnp.zeros((N_NODES, D), jnp.float32)
    h = embeds
    layer_sum = embeds
    for _ in range(3):
        partials = _propagate(h, epk, wpk, zeros)
        h, layer_sum = _combine(partials[0], partials[1], layer_sum)
    return _score(layer_sum, u_idx, v_idx)
